# baseline (speedup 1.0000x reference)
import numpy as np
import jax
import jax.numpy as jnp
from jax import lax
from jax.experimental import pallas as pl
from jax.experimental.pallas import tpu as pltpu
from jax.experimental.pallas import tpu_sc as plsc

N = 10000
E = 320000
D = 128
R = 50
H = D // 2

NC = 2
NS = 16
LANES = 16

CHUNK = 80
EDGES_PER_TILE = E // NS
CHUNKS_PER_TILE = EDGES_PER_TILE // CHUNK
DUMP_TILES = 10
DUMP_ROWS = N // DUMP_TILES
ZROWS = 100
K = 16

_MNODES = 64
_m = np.arange(_MNODES)
_unodes = np.cos(np.pi * (_m + 0.5) / _MNODES)
_Tk = np.cos(np.outer(np.arange(K), np.arccos(_unodes)))
_Pc = _Tk * (2.0 / _MNODES)
_Pc[0] *= 0.5
_Conv = np.zeros((K, K))
_c0 = np.zeros(K); _c0[0] = 1.0
_c1 = np.zeros(K); _c1[1] = 1.0
_Conv[:, 0] = _c0
_Conv[:, 1] = _c1
_prev, _cur = _c0, _c1
for _k in range(2, K):
  _nxt = 2.0 * np.roll(_cur, 1) - _prev
  _nxt[0] = -_prev[0]
  _Conv[:, _k] = _nxt
  _prev, _cur = _cur, _nxt
_TNODES = ((_unodes + 1.0) / 2.0).reshape(_MNODES, 1)

_PC32 = np.asarray(_Pc, dtype=np.float32)
_CONV32 = np.asarray(_Conv, dtype=np.float32)
_TN32 = np.asarray(_TNODES, dtype=np.float32)


def _sc_body(emb2_hbm, src_hbm, dst_hbm, typ_hbm, ts_hbm, rel2_hbm,
             agg_out, deg_out,
             src_a, dst_a, typ_a, ts_a, rows_a, rel_a,
             src_b, dst_b, typ_b, ts_b, rows_b, rel_b,
             pow_v, zagg_v, zdeg_v, agg_s, deg_s,
             sem_a1, sem_a2, sem_b1, sem_b2):
  c = lax.axis_index("c")
  s = lax.axis_index("s")

  iota = lax.iota(jnp.int32, LANES)
  zero16 = jnp.zeros((LANES,), jnp.float32)
  one16 = jnp.ones((LANES,), jnp.float32)
  m1 = (iota & 1) > 0
  m2 = (iota & 2) > 0
  m4 = (iota & 4) > 0
  m8 = (iota & 8) > 0

  def fill_zagg(i, _):
    for j in range(H // LANES):
      zagg_v[i, pl.ds(j * LANES, LANES)] = zero16
    return ()
  lax.fori_loop(0, ZROWS, fill_zagg, (), unroll=False)

  def fill_zdeg(i, _):
    zdeg_v[i, :] = zero16
    return ()
  lax.fori_loop(0, ZROWS, fill_zdeg, (), unroll=False)

  row0 = s * DUMP_ROWS

  @pl.when(s < DUMP_TILES)
  def _zero():
    for k in range(DUMP_ROWS // ZROWS):
      pltpu.sync_copy(zagg_v, agg_s.at[pl.ds(row0 + k * ZROWS, ZROWS)])
      pltpu.sync_copy(zdeg_v, deg_s.at[pl.ds(row0 + k * ZROWS, ZROWS)])

  plsc.subcore_barrier()

  tile_base = s * EDGES_PER_TILE
  src_off = c * N
  typ_off = c * R

  def load_idx(chunk, src_v, dst_v, typ_v, ts_v):
    base = tile_base + chunk * CHUNK
    pltpu.sync_copy(src_hbm.at[pl.ds(base, CHUNK)], src_v)
    pltpu.sync_copy(dst_hbm.at[pl.ds(base, CHUNK)], dst_v)
    pltpu.sync_copy(typ_hbm.at[pl.ds(base, CHUNK)], typ_v)
    pltpu.sync_copy(ts_hbm.at[pl.ds(base, CHUNK)], ts_v)
    for k in range(CHUNK // LANES):
      sl = pl.ds(k * LANES, LANES)
      src_v[sl] = src_v[sl] + src_off
      typ_v[sl] = typ_v[sl] + typ_off

  def issue(src_v, typ_v, rows_v, rel_v, s1, s2):
    pltpu.async_copy(emb2_hbm.at[src_v], rows_v, s1)
    pltpu.async_copy(rel2_hbm.at[typ_v], rel_v, s2)

  def drain(src_v, typ_v, rows_v, rel_v, s1, s2):
    pltpu.make_async_copy(emb2_hbm.at[src_v], rows_v, s1).wait()
    pltpu.make_async_copy(rel2_hbm.at[typ_v], rel_v, s2).wait()

  def moments(ts_v):
    for g in range(CHUNK // LANES):
      ts16 = ts_v[pl.ds(g * LANES, LANES)]

      def mom_body(l, _):
        tsb = _bcast(ts16, l)
        u = 2.0 * tsb - one16
        u2 = u * u
        u4 = u2 * u2
        u8 = u4 * u4
        p = jnp.where(m1, u, one16) * jnp.where(m2, u2, one16)
        p = p * jnp.where(m4, u4, one16)
        p = p * jnp.where(m8, u8, one16)
        pow_v[g * LANES + l, :] = p
        return ()

      lax.fori_loop(0, LANES, mom_body, (), unroll=4)

  def mult(rows_v, rel_v):
    def edge_body(e, _):
      for j in range(H // LANES):
        sl = pl.ds(j * LANES, LANES)
        rows_v[e, sl] = rows_v[e, sl] * rel_v[e, sl]
      return ()

    lax.fori_loop(0, CHUNK, edge_body, (), unroll=8)

  def scatter(rows_v, dst_v, with_pow):
    pltpu.sync_copy(rows_v, agg_s.at[dst_v], add=True)

    @pl.when(c == with_pow)
    def _mom_scatter():
      pltpu.sync_copy(pow_v, deg_s.at[dst_v], add=True)

  def process(src_v, dst_v, typ_v, ts_v, rows_v, rel_v, s1, s2, parity):
    @pl.when(c == parity)
    def _m():
      moments(ts_v)
    drain(src_v, typ_v, rows_v, rel_v, s1, s2)
    mult(rows_v, rel_v)
    scatter(rows_v, dst_v, parity)

  load_idx(0, src_a, dst_a, typ_a, ts_a)
  issue(src_a, typ_a, rows_a, rel_a, sem_a1, sem_a2)

  def pair_body(i, _):
    load_idx(2 * i + 1, src_b, dst_b, typ_b, ts_b)
    issue(src_b, typ_b, rows_b, rel_b, sem_b1, sem_b2)
    process(src_a, dst_a, typ_a, ts_a, rows_a, rel_a, sem_a1, sem_a2, 0)
    load_idx(2 * i + 2, src_a, dst_a, typ_a, ts_a)
    issue(src_a, typ_a, rows_a, rel_a, sem_a1, sem_a2)
    process(src_b, dst_b, typ_b, ts_b, rows_b, rel_b, sem_b1, sem_b2, 1)
    return ()

  lax.fori_loop(0, CHUNKS_PER_TILE // 2 - 1, pair_body, (), unroll=False)

  load_idx(CHUNKS_PER_TILE - 1, src_b, dst_b, typ_b, ts_b)
  issue(src_b, typ_b, rows_b, rel_b, sem_b1, sem_b2)
  process(src_a, dst_a, typ_a, ts_a, rows_a, rel_a, sem_a1, sem_a2, 0)
  process(src_b, dst_b, typ_b, ts_b, rows_b, rel_b, sem_b1, sem_b2, 1)

  plsc.subcore_barrier()

  @pl.when(s < DUMP_TILES)
  def _dump():
    pltpu.sync_copy(agg_s.at[pl.ds(row0, DUMP_ROWS)],
                    agg_out.at[pl.ds(c * N + row0, DUMP_ROWS)])
    pltpu.sync_copy(deg_s.at[pl.ds(row0, DUMP_ROWS)],
                    deg_out.at[pl.ds(c * N + row0, DUMP_ROWS)])


def _bcast(vec, l):
  idx = jnp.full((LANES, 1), l, dtype=jnp.int32)
  return lax.gather(
      vec, idx,
      lax.GatherDimensionNumbers(offset_dims=(),
                                 collapsed_slice_dims=(0,),
                                 start_index_map=(0,)),
      slice_sizes=(1,),
      mode=lax.GatherScatterMode.PROMISE_IN_BOUNDS)


@jax.jit
def _sc_edge_phase(emb2, src, dst, typ, ts, rel2):
  mesh = plsc.VectorSubcoreMesh(
      core_axis_name="c", subcore_axis_name="s", num_cores=NC,
      num_subcores=NS)
  f = pl.kernel(
      _sc_body,
      out_type=(
          jax.ShapeDtypeStruct((NC * N, H), jnp.float32),
          jax.ShapeDtypeStruct((NC * N, K), jnp.float32),
      ),
      mesh=mesh,
      scratch_types=[
          pltpu.VMEM((CHUNK,), jnp.int32),
          pltpu.VMEM((CHUNK,), jnp.int32),
          pltpu.VMEM((CHUNK,), jnp.int32),
          pltpu.VMEM((CHUNK,), jnp.float32),
          pltpu.VMEM((CHUNK, H), jnp.float32),
          pltpu.VMEM((CHUNK, H), jnp.float32),
          pltpu.VMEM((CHUNK,), jnp.int32),
          pltpu.VMEM((CHUNK,), jnp.int32),
          pltpu.VMEM((CHUNK,), jnp.int32),
          pltpu.VMEM((CHUNK,), jnp.float32),
          pltpu.VMEM((CHUNK, H), jnp.float32),
          pltpu.VMEM((CHUNK, H), jnp.float32),
          pltpu.VMEM((CHUNK, K), jnp.float32),
          pltpu.VMEM((ZROWS, H), jnp.float32),
          pltpu.VMEM((ZROWS, K), jnp.float32),
          pltpu.VMEM_SHARED((N, H), jnp.float32),
          pltpu.VMEM_SHARED((N, K), jnp.float32),
          pltpu.SemaphoreType.DMA,
          pltpu.SemaphoreType.DMA,
          pltpu.SemaphoreType.DMA,
          pltpu.SemaphoreType.DMA,
      ],
      compiler_params=pltpu.CompilerParams(use_tc_tiling_on_sc=False),
  )
  return f(emb2, src, dst, typ, ts, rel2)


def _prep_body(coeff_ref, basis_ref, w_ref, b_ref, tn_ref, pc_ref, conv_ref,
               rel_ref, a_ref):
  rel_ref[...] = jnp.dot(coeff_ref[...], basis_ref[...],
                         preferred_element_type=jnp.float32)
  cmat = jnp.cos(tn_ref[...] * w_ref[...] + b_ref[...])
  acheb = jnp.dot(pc_ref[...], cmat, preferred_element_type=jnp.float32)
  a_ref[...] = jnp.dot(conv_ref[...], acheb,
                       preferred_element_type=jnp.float32)


@jax.jit
def _prep_tc(coeff, basis, w_t, b_t):
  return pl.pallas_call(
      _prep_body,
      out_shape=(
          jax.ShapeDtypeStruct((R, D), jnp.float32),
          jax.ShapeDtypeStruct((K, D), jnp.float32),
      ),
  )(coeff, basis, w_t.reshape(1, D), b_t.reshape(1, D), _TN32, _PC32,
    _CONV32)


BLK = 1000


def _final_body(agg_ref, mom_ref, a_ref, emb_ref, w_ref, ws_ref, bias_ref,
                res_ref, out_ref):
  mom = mom_ref[0] + mom_ref[1]
  deg = jnp.maximum(mom[:, 0:1], 1.0)
  cosp = jnp.dot(mom, a_ref[...], preferred_element_type=jnp.float32)
  aggl = (agg_ref[0] + cosp[:, 0:H]) / deg
  aggr = (agg_ref[1] + cosp[:, H:D]) / deg
  emb = emb_ref[...]
  conv = jnp.tanh(
      jnp.dot(aggl, w_ref[0:H, :], preferred_element_type=jnp.float32)
      + jnp.dot(aggr, w_ref[H:D, :], preferred_element_type=jnp.float32)
      + jnp.dot(emb, ws_ref[...], preferred_element_type=jnp.float32)
      + bias_ref[...])
  out_ref[...] = emb + res_ref[...] * conv


@jax.jit
def _final_tc(agg_part, mom, a_fit, emb, W, W_self, bias_c, res):
  bias2 = bias_c.reshape(1, D)
  res2 = jnp.broadcast_to(res, (1, D)).astype(jnp.float32)
  return pl.pallas_call(
      _final_body,
      grid=(N // BLK,),
      in_specs=[
          pl.BlockSpec((NC, BLK, H), lambda i: (0, i, 0)),
          pl.BlockSpec((NC, BLK, K), lambda i: (0, i, 0)),
          pl.BlockSpec((K, D), lambda i: (0, 0)),
          pl.BlockSpec((BLK, D), lambda i: (i, 0)),
          pl.BlockSpec((D, D), lambda i: (0, 0)),
          pl.BlockSpec((D, D), lambda i: (0, 0)),
          pl.BlockSpec((1, D), lambda i: (0, 0)),
          pl.BlockSpec((1, D), lambda i: (0, 0)),
      ],
      out_specs=pl.BlockSpec((BLK, D), lambda i: (i, 0)),
      out_shape=jax.ShapeDtypeStruct((N, D), jnp.float32),
  )(agg_part, mom, a_fit, emb, W, W_self, bias2, res2)


def kernel(emb, edge_index, edge_type, edge_ts, w_t, b_t, basis, coeff, W,
           W_self, bias_c, res):
  src = edge_index[0]
  dst = edge_index[1]
  rel_emb, a_fit = _prep_tc(coeff, basis, w_t, b_t)
  emb2 = jnp.concatenate([emb[:, :H], emb[:, H:]], axis=0)
  rel2 = jnp.concatenate([rel_emb[:, :H], rel_emb[:, H:]], axis=0)
  agg_flat, mom_flat = _sc_edge_phase(emb2, src, dst, edge_type, edge_ts, rel2)
  agg_part = agg_flat.reshape(NC, N, H)
  mom_part = mom_flat.reshape(NC, N, K)
  return _final_tc(agg_part, mom_part, a_fit, emb, W, W_self, bias_c, res)


# speedup vs baseline: 2.1144x; 2.1144x over previous
import numpy as np
import jax
import jax.numpy as jnp
from jax import lax
from jax.experimental import pallas as pl
from jax.experimental.pallas import tpu as pltpu
from jax.experimental.pallas import tpu_sc as plsc

N = 10000
E = 320000
D = 128
R = 50
H = D // 2

NC = 2
NS = 16
LANES = 16

CHUNK = 80
EDGES_PER_TILE = E // NS
CHUNKS_PER_TILE = EDGES_PER_TILE // CHUNK
DUMP_TILES = 10
DUMP_ROWS = N // DUMP_TILES
ZROWS = 100
K = 16

_MNODES = 64
_m = np.arange(_MNODES)
_unodes = np.cos(np.pi * (_m + 0.5) / _MNODES)
_Tk = np.cos(np.outer(np.arange(K), np.arccos(_unodes)))
_Pc = _Tk * (2.0 / _MNODES)
_Pc[0] *= 0.5
_Conv = np.zeros((K, K))
_c0 = np.zeros(K); _c0[0] = 1.0
_c1 = np.zeros(K); _c1[1] = 1.0
_Conv[:, 0] = _c0
_Conv[:, 1] = _c1
_prev, _cur = _c0, _c1
for _k in range(2, K):
  _nxt = 2.0 * np.roll(_cur, 1) - _prev
  _nxt[0] = -_prev[0]
  _Conv[:, _k] = _nxt
  _prev, _cur = _cur, _nxt
_TNODES = ((_unodes + 1.0) / 2.0).reshape(_MNODES, 1)

_PC32 = np.asarray(_Pc, dtype=np.float32)
_CONV32 = np.asarray(_Conv, dtype=np.float32)
_TN32 = np.asarray(_TNODES, dtype=np.float32)


def _sc_body(emb2_hbm, src_hbm, dst_hbm, typ_hbm, ts_hbm, rel2_hbm,
             agg_out, deg_out,
             src_a, dst_a, typ_a, ts_a, rows_a, rel_a,
             src_b, dst_b, typ_b, ts_b, rows_b, rel_b,
             pow_v, zagg_v, zdeg_v, agg_s, deg_s,
             sem_a1, sem_a2, sem_b1, sem_b2, sem_ia, sem_ib):
  c = lax.axis_index("c")
  s = lax.axis_index("s")

  iota = lax.iota(jnp.int32, LANES)
  zero16 = jnp.zeros((LANES,), jnp.float32)
  one16 = jnp.ones((LANES,), jnp.float32)
  m1 = (iota & 1) > 0
  m2 = (iota & 2) > 0
  m4 = (iota & 4) > 0
  m8 = (iota & 8) > 0

  def fill_zagg(i, _):
    for j in range(H // LANES):
      zagg_v[i, pl.ds(j * LANES, LANES)] = zero16
    return ()
  lax.fori_loop(0, ZROWS, fill_zagg, (), unroll=False)

  def fill_zdeg(i, _):
    zdeg_v[i, :] = zero16
    return ()
  lax.fori_loop(0, ZROWS, fill_zdeg, (), unroll=False)

  row0 = s * DUMP_ROWS

  @pl.when(s < DUMP_TILES)
  def _zero():
    for k in range(DUMP_ROWS // ZROWS):
      pltpu.sync_copy(zagg_v, agg_s.at[pl.ds(row0 + k * ZROWS, ZROWS)])
      pltpu.sync_copy(zdeg_v, deg_s.at[pl.ds(row0 + k * ZROWS, ZROWS)])

  plsc.subcore_barrier()

  tile_base = s * EDGES_PER_TILE
  src_off = c * N
  typ_off = c * R

  def idx_prefetch(chunk, src_v, dst_v, typ_v, ts_v, sem):
    base = tile_base + chunk * CHUNK
    pltpu.async_copy(src_hbm.at[pl.ds(base, CHUNK)], src_v, sem)
    pltpu.async_copy(dst_hbm.at[pl.ds(base, CHUNK)], dst_v, sem)
    pltpu.async_copy(typ_hbm.at[pl.ds(base, CHUNK)], typ_v, sem)
    pltpu.async_copy(ts_hbm.at[pl.ds(base, CHUNK)], ts_v, sem)

  def idx_ready(src_v, dst_v, typ_v, ts_v, sem):
    pltpu.make_async_copy(src_hbm.at[pl.ds(0, CHUNK)], src_v, sem).wait()
    pltpu.make_async_copy(dst_hbm.at[pl.ds(0, CHUNK)], dst_v, sem).wait()
    pltpu.make_async_copy(typ_hbm.at[pl.ds(0, CHUNK)], typ_v, sem).wait()
    pltpu.make_async_copy(ts_hbm.at[pl.ds(0, CHUNK)], ts_v, sem).wait()
    for k in range(CHUNK // LANES):
      sl = pl.ds(k * LANES, LANES)
      src_v[sl] = src_v[sl] + src_off
      typ_v[sl] = typ_v[sl] + typ_off

  def issue(src_v, typ_v, rows_v, rel_v, s1, s2):
    pltpu.async_copy(emb2_hbm.at[src_v], rows_v, s1)
    pltpu.async_copy(rel2_hbm.at[typ_v], rel_v, s2)

  def drain(src_v, typ_v, rows_v, rel_v, s1, s2):
    pltpu.make_async_copy(emb2_hbm.at[src_v], rows_v, s1).wait()
    pltpu.make_async_copy(rel2_hbm.at[typ_v], rel_v, s2).wait()

  def moments(ts_v):
    for g in range(CHUNK // LANES):
      ts16 = ts_v[pl.ds(g * LANES, LANES)]

      def mom_body(l, _):
        tsb = _bcast(ts16, l)
        u = 2.0 * tsb - one16
        u2 = u * u
        u4 = u2 * u2
        u8 = u4 * u4
        p = jnp.where(m1, u, one16) * jnp.where(m2, u2, one16)
        p = p * jnp.where(m4, u4, one16)
        p = p * jnp.where(m8, u8, one16)
        pow_v[g * LANES + l, :] = p
        return ()

      lax.fori_loop(0, LANES, mom_body, (), unroll=False)

  def mult(rows_v, rel_v):
    def edge_body(e, _):
      for j in range(H // LANES):
        sl = pl.ds(j * LANES, LANES)
        rows_v[e, sl] = rows_v[e, sl] * rel_v[e, sl]
      return ()

    lax.fori_loop(0, CHUNK, edge_body, (), unroll=False)

  def scatter(rows_v, dst_v, with_pow):
    pltpu.sync_copy(rows_v, agg_s.at[dst_v], add=True)

    @pl.when(c == with_pow)
    def _mom_scatter():
      pltpu.sync_copy(pow_v, deg_s.at[dst_v], add=True)

  def process(src_v, dst_v, typ_v, ts_v, rows_v, rel_v, s1, s2, parity):
    @pl.when(c == parity)
    def _m():
      moments(ts_v)
    drain(src_v, typ_v, rows_v, rel_v, s1, s2)
    mult(rows_v, rel_v)
    scatter(rows_v, dst_v, parity)

  idx_prefetch(0, src_a, dst_a, typ_a, ts_a, sem_ia)
  idx_ready(src_a, dst_a, typ_a, ts_a, sem_ia)
  issue(src_a, typ_a, rows_a, rel_a, sem_a1, sem_a2)
  idx_prefetch(1, src_b, dst_b, typ_b, ts_b, sem_ib)

  def pair_body(i, _):
    idx_ready(src_b, dst_b, typ_b, ts_b, sem_ib)
    issue(src_b, typ_b, rows_b, rel_b, sem_b1, sem_b2)
    process(src_a, dst_a, typ_a, ts_a, rows_a, rel_a, sem_a1, sem_a2, 0)
    idx_prefetch(2 * i + 2, src_a, dst_a, typ_a, ts_a, sem_ia)
    process(src_b, dst_b, typ_b, ts_b, rows_b, rel_b, sem_b1, sem_b2, 1)
    idx_ready(src_a, dst_a, typ_a, ts_a, sem_ia)
    issue(src_a, typ_a, rows_a, rel_a, sem_a1, sem_a2)
    idx_prefetch(2 * i + 3, src_b, dst_b, typ_b, ts_b, sem_ib)
    return ()

  lax.fori_loop(0, CHUNKS_PER_TILE // 2 - 1, pair_body, (), unroll=False)

  idx_ready(src_b, dst_b, typ_b, ts_b, sem_ib)
  issue(src_b, typ_b, rows_b, rel_b, sem_b1, sem_b2)
  process(src_a, dst_a, typ_a, ts_a, rows_a, rel_a, sem_a1, sem_a2, 0)
  process(src_b, dst_b, typ_b, ts_b, rows_b, rel_b, sem_b1, sem_b2, 1)

  plsc.subcore_barrier()

  @pl.when(s < DUMP_TILES)
  def _dump():
    pltpu.sync_copy(agg_s.at[pl.ds(row0, DUMP_ROWS)],
                    agg_out.at[pl.ds(c * N + row0, DUMP_ROWS)])
    pltpu.sync_copy(deg_s.at[pl.ds(row0, DUMP_ROWS)],
                    deg_out.at[pl.ds(c * N + row0, DUMP_ROWS)])


def _bcast(vec, l):
  idx = jnp.full((LANES, 1), l, dtype=jnp.int32)
  return lax.gather(
      vec, idx,
      lax.GatherDimensionNumbers(offset_dims=(),
                                 collapsed_slice_dims=(0,),
                                 start_index_map=(0,)),
      slice_sizes=(1,),
      mode=lax.GatherScatterMode.PROMISE_IN_BOUNDS)


@jax.jit
def _sc_edge_phase(emb2, src, dst, typ, ts, rel2):
  mesh = plsc.VectorSubcoreMesh(
      core_axis_name="c", subcore_axis_name="s", num_cores=NC,
      num_subcores=NS)
  f = pl.kernel(
      _sc_body,
      out_type=(
          jax.ShapeDtypeStruct((NC * N, H), jnp.float32),
          jax.ShapeDtypeStruct((NC * N, K), jnp.float32),
      ),
      mesh=mesh,
      scratch_types=[
          pltpu.VMEM((CHUNK,), jnp.int32),
          pltpu.VMEM((CHUNK,), jnp.int32),
          pltpu.VMEM((CHUNK,), jnp.int32),
          pltpu.VMEM((CHUNK,), jnp.float32),
          pltpu.VMEM((CHUNK, H), jnp.float32),
          pltpu.VMEM((CHUNK, H), jnp.float32),
          pltpu.VMEM((CHUNK,), jnp.int32),
          pltpu.VMEM((CHUNK,), jnp.int32),
          pltpu.VMEM((CHUNK,), jnp.int32),
          pltpu.VMEM((CHUNK,), jnp.float32),
          pltpu.VMEM((CHUNK, H), jnp.float32),
          pltpu.VMEM((CHUNK, H), jnp.float32),
          pltpu.VMEM((CHUNK, K), jnp.float32),
          pltpu.VMEM((ZROWS, H), jnp.float32),
          pltpu.VMEM((ZROWS, K), jnp.float32),
          pltpu.VMEM_SHARED((N, H), jnp.float32),
          pltpu.VMEM_SHARED((N, K), jnp.float32),
          pltpu.SemaphoreType.DMA,
          pltpu.SemaphoreType.DMA,
          pltpu.SemaphoreType.DMA,
          pltpu.SemaphoreType.DMA,
          pltpu.SemaphoreType.DMA,
          pltpu.SemaphoreType.DMA,
      ],
      compiler_params=pltpu.CompilerParams(use_tc_tiling_on_sc=False),
  )
  return f(emb2, src, dst, typ, ts, rel2)


def _prep_body(coeff_ref, basis_ref, w_ref, b_ref, tn_ref, pc_ref, conv_ref,
               rel_ref, a_ref):
  rel_ref[...] = jnp.dot(coeff_ref[...], basis_ref[...],
                         preferred_element_type=jnp.float32)
  cmat = jnp.cos(tn_ref[...] * w_ref[...] + b_ref[...])
  acheb = jnp.dot(pc_ref[...], cmat, preferred_element_type=jnp.float32)
  a_ref[...] = jnp.dot(conv_ref[...], acheb,
                       preferred_element_type=jnp.float32)


@jax.jit
def _prep_tc(coeff, basis, w_t, b_t):
  return pl.pallas_call(
      _prep_body,
      out_shape=(
          jax.ShapeDtypeStruct((R, D), jnp.float32),
          jax.ShapeDtypeStruct((K, D), jnp.float32),
      ),
  )(coeff, basis, w_t.reshape(1, D), b_t.reshape(1, D), _TN32, _PC32,
    _CONV32)


BLK = 1000


def _final_body(agg_ref, mom_ref, a_ref, emb_ref, w_ref, ws_ref, bias_ref,
                res_ref, out_ref):
  mom = mom_ref[0] + mom_ref[1]
  deg = jnp.maximum(mom[:, 0:1], 1.0)
  cosp = jnp.dot(mom, a_ref[...], preferred_element_type=jnp.float32)
  aggl = (agg_ref[0] + cosp[:, 0:H]) / deg
  aggr = (agg_ref[1] + cosp[:, H:D]) / deg
  emb = emb_ref[...]
  conv = jnp.tanh(
      jnp.dot(aggl, w_ref[0:H, :], preferred_element_type=jnp.float32)
      + jnp.dot(aggr, w_ref[H:D, :], preferred_element_type=jnp.float32)
      + jnp.dot(emb, ws_ref[...], preferred_element_type=jnp.float32)
      + bias_ref[...])
  out_ref[...] = emb + res_ref[...] * conv


@jax.jit
def _final_tc(agg_part, mom, a_fit, emb, W, W_self, bias_c, res):
  bias2 = bias_c.reshape(1, D)
  res2 = jnp.broadcast_to(res, (1, D)).astype(jnp.float32)
  return pl.pallas_call(
      _final_body,
      grid=(N // BLK,),
      in_specs=[
          pl.BlockSpec((NC, BLK, H), lambda i: (0, i, 0)),
          pl.BlockSpec((NC, BLK, K), lambda i: (0, i, 0)),
          pl.BlockSpec((K, D), lambda i: (0, 0)),
          pl.BlockSpec((BLK, D), lambda i: (i, 0)),
          pl.BlockSpec((D, D), lambda i: (0, 0)),
          pl.BlockSpec((D, D), lambda i: (0, 0)),
          pl.BlockSpec((1, D), lambda i: (0, 0)),
          pl.BlockSpec((1, D), lambda i: (0, 0)),
      ],
      out_specs=pl.BlockSpec((BLK, D), lambda i: (i, 0)),
      out_shape=jax.ShapeDtypeStruct((N, D), jnp.float32),
  )(agg_part, mom, a_fit, emb, W, W_self, bias2, res2)


def kernel(emb, edge_index, edge_type, edge_ts, w_t, b_t, basis, coeff, W,
           W_self, bias_c, res):
  src = edge_index[0]
  dst = edge_index[1]
  rel_emb, a_fit = _prep_tc(coeff, basis, w_t, b_t)
  emb2 = jnp.concatenate([emb[:, :H], emb[:, H:]], axis=0)
  rel2 = jnp.concatenate([rel_emb[:, :H], rel_emb[:, H:]], axis=0)
  agg_flat, mom_flat = _sc_edge_phase(emb2, src, dst, edge_type, edge_ts, rel2)
  agg_part = agg_flat.reshape(NC, N, H)
  mom_part = mom_flat.reshape(NC, N, K)
  return _final_tc(agg_part, mom_part, a_fit, emb, W, W_self, bias_c, res)
